# baseline (speedup 1.0000x reference)
import functools

import jax
import jax.numpy as jnp
from jax import lax
from jax.experimental import pallas as pl
from jax.experimental.pallas import tpu as pltpu
from jax.experimental.pallas import tpu_sc as plsc

_B, _V, _F = 64, 512, 128
_NS, _NLR, _K = 4, 32, 16
_NSLR = _NS + _NLR
_NAGG = 2 * _NLR


def _phase_a_body(x_ref, wx_ref, wm_ref, b_ref, s_ref, lr_ref):
    xb = x_ref[0]
    xm = jnp.mean(xb, axis=0, keepdims=True)
    y = jnp.dot(xb, wx_ref[...], preferred_element_type=jnp.float32)
    y = y + jnp.dot(xm, wm_ref[...], preferred_element_type=jnp.float32)
    y = jnp.maximum(y + b_ref[...], 0.0)
    s_ref[0] = y[:, :_NS]
    lr_ref[0] = y[:, _NS:]


def _phase_a(x, wx, wm, b):
    return pl.pallas_call(
        _phase_a_body,
        grid=(_B,),
        in_specs=[
            pl.BlockSpec((1, _V, _F), lambda i: (i, 0, 0)),
            pl.BlockSpec((_F, _NSLR), lambda i: (0, 0)),
            pl.BlockSpec((_F, _NSLR), lambda i: (0, 0)),
            pl.BlockSpec((1, _NSLR), lambda i: (0, 0)),
        ],
        out_specs=[
            pl.BlockSpec((1, _V, _NS), lambda i: (i, 0, 0)),
            pl.BlockSpec((1, _V, _NLR), lambda i: (i, 0, 0)),
        ],
        out_shape=[
            jax.ShapeDtypeStruct((_B, _V, _NS), jnp.float32),
            jax.ShapeDtypeStruct((_B, _V, _NLR), jnp.float32),
        ],
    )(x, wx, wm, b)


def _phase_b_body(s_hbm, lr_hbm, agg_hbm,
                  s_v, lr_v, out_v, dbuf_v,
                  bltj_v, bltd_v, beqj_v):
    info = plsc.get_sparse_core_info()
    nc = info.num_cores
    wid = lax.axis_index("s") * nc + lax.axis_index("c")
    iota = jnp.arange(16, dtype=jnp.int32)
    inf16 = jnp.full((16,), jnp.inf, jnp.float32)
    zero16i = jnp.zeros((16,), jnp.int32)

    def one_batch(b):
        pltpu.sync_copy(s_hbm.at[b], s_v.at[pl.ds(0, _V * _NS)])
        pltpu.sync_copy(lr_hbm.at[b], lr_v.at[pl.ds(0, _V * _NLR)])

        def finish_row(i, best_d, best_j, d17v, dbase):
            t = jnp.max(best_d)
            t17 = jnp.min(d17v)

            def fixup(_):
                c_lt = jnp.sum((best_d < t).astype(jnp.int32))

                def fchunk(c, offs):
                    d = dbuf_v[pl.ds(dbase + c * 16, 16)]
                    jv = iota + c * 16

                    def compact(offs):
                        off_lt, off_eq = offs
                        mlt = d < t
                        meq = d == t
                        plsc.store_compressed(bltj_v.at[pl.ds(off_lt, 16)],
                                              jv, mask=mlt)
                        plsc.store_compressed(bltd_v.at[pl.ds(off_lt, 16)],
                                              d, mask=mlt)
                        plsc.store_compressed(beqj_v.at[pl.ds(off_eq, 16)],
                                              jv, mask=meq)
                        off_lt = off_lt + jnp.sum(mlt.astype(jnp.int32))
                        off_eq = off_eq + jnp.sum(meq.astype(jnp.int32))
                        return off_lt, off_eq

                    cnt = plsc.all_reduce_population_count(d <= t)
                    return lax.cond(cnt[0] > 0, compact, lambda oo: oo, offs)

                lax.fori_loop(0, 32, fchunk,
                              (jnp.int32(0), jnp.int32(0)))
                mk = iota < c_lt
                ja = plsc.load_gather(bltj_v, [iota])
                da = plsc.load_gather(bltd_v, [iota])
                jb = plsc.load_gather(
                    beqj_v, [jnp.maximum(iota - c_lt, 0)])
                return (jnp.where(mk, ja, jb),
                        jnp.where(mk, da, t))

            j_sel, d_sel = lax.cond(
                t17 == t, fixup, lambda _: (best_j, best_d), None)
            w = jnp.exp(-10.0 * d_sel)

            z = jnp.zeros((16,), jnp.float32)
            m0, m1, x0, x1 = z, z, z, z
            for k in range(16):
                jk = j_sel[k]
                wk = w[k]
                r0 = wk * lr_v[pl.ds(jk * _NLR, 16)]
                r1 = wk * lr_v[pl.ds(jk * _NLR + 16, 16)]
                m0 = m0 + r0
                m1 = m1 + r1
                x0 = jnp.maximum(x0, r0)
                x1 = jnp.maximum(x1, r1)
            out_v[pl.ds(i * _NAGG, 16)] = m0
            out_v[pl.ds(i * _NAGG + 16, 16)] = m1
            out_v[pl.ds(i * _NAGG + 32, 16)] = x0
            out_v[pl.ds(i * _NAGG + 48, 16)] = x1

        n_rows = 4

        def row_group(p, _):
            i0 = p * n_rows
            srows = [s_v[pl.ds((i0 + 4 * q) * _NS, 16)]
                     for q in range(n_rows // 4)]
            si = [[srows[r // 4][(r % 4) * _NS + c] for c in range(_NS)]
                  for r in range(n_rows)]

            def chunk(c, carry):
                jv = iota + c * 16
                jv4 = jv * _NS
                g = [plsc.load_gather(s_v, [jv4 + cc]) for cc in range(_NS)]
                nxt = []
                for r in range(n_rows):
                    bd, bj, d17 = carry[3 * r:3 * r + 3]
                    e = [g[cc] - si[r][cc] for cc in range(_NS)]
                    d = e[0] * e[0] + e[1] * e[1] + e[2] * e[2] + e[3] * e[3]
                    dbuf_v[pl.ds(r * _V + c * 16, 16)] = d
                    ds, js = plsc.sort_key_val(d, jv)
                    rb = lax.rev(bd, (0,))
                    rbj = lax.rev(bj, (0,))
                    take = ds < rb
                    nd = jnp.where(take, ds, rb)
                    nj = jnp.where(take, js, rbj)
                    d17 = jnp.minimum(d17, jnp.where(take, rb, ds))
                    bd, bj = plsc.sort_key_val(nd, nj)
                    nxt += [bd, bj, d17]
                return tuple(nxt)

            res = lax.fori_loop(
                0, 32, chunk, (inf16, zero16i, inf16) * n_rows,
                unroll=2)
            for r in range(n_rows):
                finish_row(i0 + r, res[3 * r], res[3 * r + 1],
                           res[3 * r + 2], r * _V)
            return 0

        lax.fori_loop(0, _V // n_rows, row_group, 0)
        pltpu.sync_copy(out_v, agg_hbm.at[b])

    for bi in range(_B // 32):
        one_batch(wid * (_B // 32) + bi)


def _phase_b(s, lr):
    mesh = plsc.VectorSubcoreMesh(core_axis_name="c", subcore_axis_name="s")
    f = pl.kernel(
        _phase_b_body,
        out_type=jax.ShapeDtypeStruct((_B, _V * _NAGG), jnp.float32),
        mesh=mesh,
        compiler_params=pltpu.CompilerParams(needs_layout_passes=False),
        scratch_types=[
            pltpu.VMEM((_V * _NS + 16,), jnp.float32),
            pltpu.VMEM((_V * _NLR + 16,), jnp.float32),
            pltpu.VMEM((_V * _NAGG,), jnp.float32),
            pltpu.VMEM((8 * _V,), jnp.float32),
            pltpu.VMEM((32,), jnp.int32),
            pltpu.VMEM((32,), jnp.float32),
            pltpu.VMEM((_V + 16,), jnp.int32),
        ],
    )
    agg = f(s.reshape(_B, _V * _NS), lr.reshape(_B, _V * _NLR))
    return agg.reshape(_B, _V, _NAGG)


def _phase_c_body(x_ref, agg_ref, wa_ref, wb_ref, wc_ref, b_ref, o_ref):
    xb = x_ref[0]
    xm = jnp.mean(xb, axis=0, keepdims=True)
    acc = jnp.dot(xb, wa_ref[...], preferred_element_type=jnp.float32)
    acc = acc + jnp.dot(xm, wb_ref[...], preferred_element_type=jnp.float32)
    acc = acc + jnp.dot(agg_ref[0], wc_ref[...],
                        preferred_element_type=jnp.float32)
    o_ref[0] = jnp.maximum(acc + b_ref[...], 0.0)


def _phase_c(x, agg, wa, wb, wc, b):
    n_out = wa.shape[1]
    return pl.pallas_call(
        _phase_c_body,
        grid=(_B,),
        in_specs=[
            pl.BlockSpec((1, _V, _F), lambda i: (i, 0, 0)),
            pl.BlockSpec((1, _V, _NAGG), lambda i: (i, 0, 0)),
            pl.BlockSpec((_F, n_out), lambda i: (0, 0)),
            pl.BlockSpec((_F, n_out), lambda i: (0, 0)),
            pl.BlockSpec((_NAGG, n_out), lambda i: (0, 0)),
            pl.BlockSpec((1, n_out), lambda i: (0, 0)),
        ],
        out_specs=pl.BlockSpec((1, _V, n_out), lambda i: (i, 0, 0)),
        out_shape=jax.ShapeDtypeStruct((_B, _V, n_out), jnp.float32),
    )(x, agg, wa, wb, wc, b)


@jax.jit
def kernel(x, W_slr, b_slr, W_out, b_out):
    f = _F
    s, lr = _phase_a(x, W_slr[:f], W_slr[f:], b_slr.reshape(1, -1))
    agg = _phase_b(s, lr)
    wc = jnp.concatenate(
        [W_out[2 * f:2 * f + _NLR] / float(_K), W_out[2 * f + _NLR:]], axis=0)
    out = _phase_c(x, agg, W_out[:f], W_out[f:2 * f], wc,
                   b_out.reshape(1, -1))
    return out


# speedup vs baseline: 1.1400x; 1.1400x over previous
import functools

import jax
import jax.numpy as jnp
from jax import lax
from jax.experimental import pallas as pl
from jax.experimental.pallas import tpu as pltpu
from jax.experimental.pallas import tpu_sc as plsc

_B, _V, _F = 64, 512, 128
_NS, _NLR, _K = 4, 32, 16
_NSLR = _NS + _NLR
_NAGG = 2 * _NLR


_BB = 4


def _phase_a_body(x_ref, wx_ref, wm_ref, b_ref, s_ref, lr_ref):
    for q in range(_BB):
        xb = x_ref[q]
        xm = jnp.mean(xb, axis=0, keepdims=True)
        y = jnp.dot(xb, wx_ref[...], preferred_element_type=jnp.float32)
        y = y + jnp.dot(xm, wm_ref[...],
                        preferred_element_type=jnp.float32)
        y = jnp.maximum(y + b_ref[...], 0.0)
        s_ref[q] = y[:, :_NS]
        lr_ref[q] = y[:, _NS:]


def _phase_a(x, wx, wm, b):
    return pl.pallas_call(
        _phase_a_body,
        grid=(_B // _BB,),
        in_specs=[
            pl.BlockSpec((_BB, _V, _F), lambda i: (i, 0, 0)),
            pl.BlockSpec((_F, _NSLR), lambda i: (0, 0)),
            pl.BlockSpec((_F, _NSLR), lambda i: (0, 0)),
            pl.BlockSpec((1, _NSLR), lambda i: (0, 0)),
        ],
        out_specs=[
            pl.BlockSpec((_BB, _V, _NS), lambda i: (i, 0, 0)),
            pl.BlockSpec((_BB, _V, _NLR), lambda i: (i, 0, 0)),
        ],
        out_shape=[
            jax.ShapeDtypeStruct((_B, _V, _NS), jnp.float32),
            jax.ShapeDtypeStruct((_B, _V, _NLR), jnp.float32),
        ],
    )(x, wx, wm, b)


def _phase_b_body(s_hbm, lr_hbm, agg_hbm,
                  s_v, lr_v, out_v, dbuf_v,
                  bltj_v, bltd_v, beqj_v):
    info = plsc.get_sparse_core_info()
    nc = info.num_cores
    wid = lax.axis_index("s") * nc + lax.axis_index("c")
    iota = jnp.arange(16, dtype=jnp.int32)
    inf16 = jnp.full((16,), jnp.inf, jnp.float32)
    zero16i = jnp.zeros((16,), jnp.int32)

    def one_batch(b):
        pltpu.sync_copy(s_hbm.at[b], s_v.at[pl.ds(0, _V * _NS)])
        pltpu.sync_copy(lr_hbm.at[b], lr_v.at[pl.ds(0, _V * _NLR)])

        def finish_row(i, best_d, best_j, d17v, dbase):
            t = jnp.max(best_d)
            t17 = jnp.min(d17v)

            def fixup(_):
                c_lt = jnp.sum((best_d < t).astype(jnp.int32))

                def fchunk(c, offs):
                    d = dbuf_v[pl.ds(dbase + c * 16, 16)]
                    jv = iota + c * 16

                    def compact(offs):
                        off_lt, off_eq = offs
                        mlt = d < t
                        meq = d == t
                        plsc.store_compressed(bltj_v.at[pl.ds(off_lt, 16)],
                                              jv, mask=mlt)
                        plsc.store_compressed(bltd_v.at[pl.ds(off_lt, 16)],
                                              d, mask=mlt)
                        plsc.store_compressed(beqj_v.at[pl.ds(off_eq, 16)],
                                              jv, mask=meq)
                        off_lt = off_lt + jnp.sum(mlt.astype(jnp.int32))
                        off_eq = off_eq + jnp.sum(meq.astype(jnp.int32))
                        return off_lt, off_eq

                    cnt = plsc.all_reduce_population_count(d <= t)
                    return lax.cond(cnt[0] > 0, compact, lambda oo: oo, offs)

                lax.fori_loop(0, 32, fchunk,
                              (jnp.int32(0), jnp.int32(0)))
                mk = iota < c_lt
                ja = plsc.load_gather(bltj_v, [iota])
                da = plsc.load_gather(bltd_v, [iota])
                jb = plsc.load_gather(
                    beqj_v, [jnp.maximum(iota - c_lt, 0)])
                return (jnp.where(mk, ja, jb),
                        jnp.where(mk, da, t))

            j_sel, d_sel = lax.cond(
                t17 == t, fixup, lambda _: (best_j, best_d), None)
            w = jnp.exp(-10.0 * d_sel)

            z = jnp.zeros((16,), jnp.float32)
            m0, m1, x0, x1 = z, z, z, z
            for k in range(16):
                jk = j_sel[k]
                wk = w[k]
                r0 = wk * lr_v[pl.ds(jk * _NLR, 16)]
                r1 = wk * lr_v[pl.ds(jk * _NLR + 16, 16)]
                m0 = m0 + r0
                m1 = m1 + r1
                x0 = jnp.maximum(x0, r0)
                x1 = jnp.maximum(x1, r1)
            out_v[pl.ds(i * _NAGG, 16)] = m0
            out_v[pl.ds(i * _NAGG + 16, 16)] = m1
            out_v[pl.ds(i * _NAGG + 32, 16)] = x0
            out_v[pl.ds(i * _NAGG + 48, 16)] = x1

        n_rows = 4

        def row_group(p, _):
            i0 = p * n_rows
            srows = [s_v[pl.ds((i0 + 4 * q) * _NS, 16)]
                     for q in range(n_rows // 4)]
            si = [[srows[r // 4][(r % 4) * _NS + c] for c in range(_NS)]
                  for r in range(n_rows)]

            def chunk(c, carry):
                jv = iota + c * 16
                jv4 = jv * _NS
                g = [plsc.load_gather(s_v, [jv4 + cc]) for cc in range(_NS)]
                nxt = []
                for r in range(n_rows):
                    bd, bj, d17 = carry[3 * r:3 * r + 3]
                    e = [g[cc] - si[r][cc] for cc in range(_NS)]
                    d = e[0] * e[0] + e[1] * e[1] + e[2] * e[2] + e[3] * e[3]
                    dbuf_v[pl.ds(r * _V + c * 16, 16)] = d
                    ds, js = plsc.sort_key_val(d, jv)
                    rb = lax.rev(bd, (0,))
                    rbj = lax.rev(bj, (0,))
                    take = ds < rb
                    nd = jnp.where(take, ds, rb)
                    nj = jnp.where(take, js, rbj)
                    d17 = jnp.minimum(d17, jnp.where(take, rb, ds))
                    bd, bj = plsc.sort_key_val(nd, nj)
                    nxt += [bd, bj, d17]
                return tuple(nxt)

            res = lax.fori_loop(
                0, 32, chunk, (inf16, zero16i, inf16) * n_rows)
            for r in range(n_rows):
                finish_row(i0 + r, res[3 * r], res[3 * r + 1],
                           res[3 * r + 2], r * _V)
            return 0

        lax.fori_loop(0, _V // n_rows, row_group, 0)
        pltpu.sync_copy(out_v, agg_hbm.at[b])

    for bi in range(_B // 32):
        one_batch(wid * (_B // 32) + bi)


def _phase_b(s, lr):
    mesh = plsc.VectorSubcoreMesh(core_axis_name="c", subcore_axis_name="s")
    f = pl.kernel(
        _phase_b_body,
        out_type=jax.ShapeDtypeStruct((_B, _V * _NAGG), jnp.float32),
        mesh=mesh,
        compiler_params=pltpu.CompilerParams(needs_layout_passes=False),
        scratch_types=[
            pltpu.VMEM((_V * _NS + 16,), jnp.float32),
            pltpu.VMEM((_V * _NLR + 16,), jnp.float32),
            pltpu.VMEM((_V * _NAGG,), jnp.float32),
            pltpu.VMEM((8 * _V,), jnp.float32),
            pltpu.VMEM((32,), jnp.int32),
            pltpu.VMEM((32,), jnp.float32),
            pltpu.VMEM((_V + 16,), jnp.int32),
        ],
    )
    agg = f(s.reshape(_B, _V * _NS), lr.reshape(_B, _V * _NLR))
    return agg.reshape(_B, _V, _NAGG)


def _phase_c_body(x_ref, agg_ref, wa_ref, wb_ref, wc_ref, b_ref, o_ref):
    for q in range(_BB):
        xb = x_ref[q]
        xm = jnp.mean(xb, axis=0, keepdims=True)
        acc = jnp.dot(xb, wa_ref[...], preferred_element_type=jnp.float32)
        acc = acc + jnp.dot(xm, wb_ref[...],
                            preferred_element_type=jnp.float32)
        acc = acc + jnp.dot(agg_ref[q], wc_ref[...],
                            preferred_element_type=jnp.float32)
        o_ref[q] = jnp.maximum(acc + b_ref[...], 0.0)


def _phase_c(x, agg, wa, wb, wc, b):
    n_out = wa.shape[1]
    return pl.pallas_call(
        _phase_c_body,
        grid=(_B // _BB,),
        in_specs=[
            pl.BlockSpec((_BB, _V, _F), lambda i: (i, 0, 0)),
            pl.BlockSpec((_BB, _V, _NAGG), lambda i: (i, 0, 0)),
            pl.BlockSpec((_F, n_out), lambda i: (0, 0)),
            pl.BlockSpec((_F, n_out), lambda i: (0, 0)),
            pl.BlockSpec((_NAGG, n_out), lambda i: (0, 0)),
            pl.BlockSpec((1, n_out), lambda i: (0, 0)),
        ],
        out_specs=pl.BlockSpec((_BB, _V, n_out), lambda i: (i, 0, 0)),
        out_shape=jax.ShapeDtypeStruct((_B, _V, n_out), jnp.float32),
    )(x, agg, wa, wb, wc, b)


@jax.jit
def kernel(x, W_slr, b_slr, W_out, b_out):
    f = _F
    s, lr = _phase_a(x, W_slr[:f], W_slr[f:], b_slr.reshape(1, -1))
    agg = _phase_b(s, lr)
    wc = jnp.concatenate(
        [W_out[2 * f:2 * f + _NLR] / float(_K), W_out[2 * f + _NLR:]], axis=0)
    out = _phase_c(x, agg, W_out[:f], W_out[f:2 * f], wc,
                   b_out.reshape(1, -1))
    return out


# speedup vs baseline: 1.1664x; 1.0232x over previous
import functools

import jax
import jax.numpy as jnp
from jax import lax
from jax.experimental import pallas as pl
from jax.experimental.pallas import tpu as pltpu
from jax.experimental.pallas import tpu_sc as plsc

_B, _V, _F = 64, 512, 128
_NS, _NLR, _K = 4, 32, 16
_NSLR = _NS + _NLR
_NAGG = 2 * _NLR


_BB = 8


def _phase_a_body(x_ref, wx_ref, wm_ref, b_ref, s_ref, lr_ref):
    for q in range(_BB):
        xb = x_ref[q]
        xm = jnp.mean(xb, axis=0, keepdims=True)
        y = jnp.dot(xb, wx_ref[...], preferred_element_type=jnp.float32)
        y = y + jnp.dot(xm, wm_ref[...],
                        preferred_element_type=jnp.float32)
        y = jnp.maximum(y + b_ref[...], 0.0)
        s_ref[q] = y[:, :_NS]
        lr_ref[q] = y[:, _NS:]


def _phase_a(x, wx, wm, b):
    return pl.pallas_call(
        _phase_a_body,
        grid=(_B // _BB,),
        in_specs=[
            pl.BlockSpec((_BB, _V, _F), lambda i: (i, 0, 0)),
            pl.BlockSpec((_F, _NSLR), lambda i: (0, 0)),
            pl.BlockSpec((_F, _NSLR), lambda i: (0, 0)),
            pl.BlockSpec((1, _NSLR), lambda i: (0, 0)),
        ],
        out_specs=[
            pl.BlockSpec((_BB, _V, _NS), lambda i: (i, 0, 0)),
            pl.BlockSpec((_BB, _V, _NLR), lambda i: (i, 0, 0)),
        ],
        out_shape=[
            jax.ShapeDtypeStruct((_B, _V, _NS), jnp.float32),
            jax.ShapeDtypeStruct((_B, _V, _NLR), jnp.float32),
        ],
    )(x, wx, wm, b)


def _phase_b_body(s_hbm, lr_hbm, agg_hbm,
                  s_v, lr_v, out_v, dbuf_v,
                  bltj_v, bltd_v, beqj_v):
    info = plsc.get_sparse_core_info()
    nc = info.num_cores
    wid = lax.axis_index("s") * nc + lax.axis_index("c")
    iota = jnp.arange(16, dtype=jnp.int32)
    inf16 = jnp.full((16,), jnp.inf, jnp.float32)
    zero16i = jnp.zeros((16,), jnp.int32)

    def one_batch(b):
        pltpu.sync_copy(s_hbm.at[b], s_v.at[pl.ds(0, _V * _NS)])
        pltpu.sync_copy(lr_hbm.at[b], lr_v.at[pl.ds(0, _V * _NLR)])

        def finish_row(i, best_d, best_j, d17v, dbase):
            t = jnp.max(best_d)
            t17 = jnp.min(d17v)

            def fixup(_):
                c_lt = jnp.sum((best_d < t).astype(jnp.int32))

                def fchunk(c, offs):
                    d = dbuf_v[pl.ds(dbase + c * 16, 16)]
                    jv = iota + c * 16

                    def compact(offs):
                        off_lt, off_eq = offs
                        mlt = d < t
                        meq = d == t
                        plsc.store_compressed(bltj_v.at[pl.ds(off_lt, 16)],
                                              jv, mask=mlt)
                        plsc.store_compressed(bltd_v.at[pl.ds(off_lt, 16)],
                                              d, mask=mlt)
                        plsc.store_compressed(beqj_v.at[pl.ds(off_eq, 16)],
                                              jv, mask=meq)
                        off_lt = off_lt + jnp.sum(mlt.astype(jnp.int32))
                        off_eq = off_eq + jnp.sum(meq.astype(jnp.int32))
                        return off_lt, off_eq

                    cnt = plsc.all_reduce_population_count(d <= t)
                    return lax.cond(cnt[0] > 0, compact, lambda oo: oo, offs)

                lax.fori_loop(0, 32, fchunk,
                              (jnp.int32(0), jnp.int32(0)))
                mk = iota < c_lt
                ja = plsc.load_gather(bltj_v, [iota])
                da = plsc.load_gather(bltd_v, [iota])
                jb = plsc.load_gather(
                    beqj_v, [jnp.maximum(iota - c_lt, 0)])
                return (jnp.where(mk, ja, jb),
                        jnp.where(mk, da, t))

            j_sel, d_sel = lax.cond(
                t17 == t, fixup, lambda _: (best_j, best_d), None)
            w = jnp.exp(-10.0 * d_sel)

            z = jnp.zeros((16,), jnp.float32)
            m0, m1, x0, x1 = z, z, z, z
            for k in range(16):
                jk = j_sel[k]
                wk = w[k]
                r0 = wk * lr_v[pl.ds(jk * _NLR, 16)]
                r1 = wk * lr_v[pl.ds(jk * _NLR + 16, 16)]
                m0 = m0 + r0
                m1 = m1 + r1
                x0 = jnp.maximum(x0, r0)
                x1 = jnp.maximum(x1, r1)
            out_v[pl.ds(i * _NAGG, 16)] = m0
            out_v[pl.ds(i * _NAGG + 16, 16)] = m1
            out_v[pl.ds(i * _NAGG + 32, 16)] = x0
            out_v[pl.ds(i * _NAGG + 48, 16)] = x1

        n_rows = 4

        def row_group(p, _):
            i0 = p * n_rows
            srows = [s_v[pl.ds((i0 + 4 * q) * _NS, 16)]
                     for q in range(n_rows // 4)]
            si = [[srows[r // 4][(r % 4) * _NS + c] for c in range(_NS)]
                  for r in range(n_rows)]

            def chunk(c, carry):
                jv = iota + c * 16
                jv4 = jv * _NS
                g = [plsc.load_gather(s_v, [jv4 + cc]) for cc in range(_NS)]
                nxt = []
                for r in range(n_rows):
                    bd, bj, d17 = carry[3 * r:3 * r + 3]
                    e = [g[cc] - si[r][cc] for cc in range(_NS)]
                    d = e[0] * e[0] + e[1] * e[1] + e[2] * e[2] + e[3] * e[3]
                    dbuf_v[pl.ds(r * _V + c * 16, 16)] = d
                    ds, js = plsc.sort_key_val(d, jv)
                    rb = lax.rev(bd, (0,))
                    rbj = lax.rev(bj, (0,))
                    take = ds < rb
                    nd = jnp.where(take, ds, rb)
                    nj = jnp.where(take, js, rbj)
                    d17 = jnp.minimum(d17, jnp.where(take, rb, ds))
                    bd, bj = plsc.sort_key_val(nd, nj)
                    nxt += [bd, bj, d17]
                return tuple(nxt)

            res = lax.fori_loop(
                0, 32, chunk, (inf16, zero16i, inf16) * n_rows)
            for r in range(n_rows):
                finish_row(i0 + r, res[3 * r], res[3 * r + 1],
                           res[3 * r + 2], r * _V)
            return 0

        lax.fori_loop(0, _V // n_rows, row_group, 0)
        pltpu.sync_copy(out_v, agg_hbm.at[b])

    for bi in range(_B // 32):
        one_batch(wid * (_B // 32) + bi)


def _phase_b(s, lr):
    mesh = plsc.VectorSubcoreMesh(core_axis_name="c", subcore_axis_name="s")
    f = pl.kernel(
        _phase_b_body,
        out_type=jax.ShapeDtypeStruct((_B, _V * _NAGG), jnp.float32),
        mesh=mesh,
        compiler_params=pltpu.CompilerParams(needs_layout_passes=False),
        scratch_types=[
            pltpu.VMEM((_V * _NS + 16,), jnp.float32),
            pltpu.VMEM((_V * _NLR + 16,), jnp.float32),
            pltpu.VMEM((_V * _NAGG,), jnp.float32),
            pltpu.VMEM((8 * _V,), jnp.float32),
            pltpu.VMEM((32,), jnp.int32),
            pltpu.VMEM((32,), jnp.float32),
            pltpu.VMEM((_V + 16,), jnp.int32),
        ],
    )
    agg = f(s.reshape(_B, _V * _NS), lr.reshape(_B, _V * _NLR))
    return agg.reshape(_B, _V, _NAGG)


def _phase_c_body(x_ref, agg_ref, wa_ref, wb_ref, wc_ref, b_ref, o_ref):
    for q in range(_BB):
        xb = x_ref[q]
        xm = jnp.mean(xb, axis=0, keepdims=True)
        acc = jnp.dot(xb, wa_ref[...], preferred_element_type=jnp.float32)
        acc = acc + jnp.dot(xm, wb_ref[...],
                            preferred_element_type=jnp.float32)
        acc = acc + jnp.dot(agg_ref[q], wc_ref[...],
                            preferred_element_type=jnp.float32)
        o_ref[q] = jnp.maximum(acc + b_ref[...], 0.0)


def _phase_c(x, agg, wa, wb, wc, b):
    n_out = wa.shape[1]
    return pl.pallas_call(
        _phase_c_body,
        grid=(_B // _BB,),
        in_specs=[
            pl.BlockSpec((_BB, _V, _F), lambda i: (i, 0, 0)),
            pl.BlockSpec((_BB, _V, _NAGG), lambda i: (i, 0, 0)),
            pl.BlockSpec((_F, n_out), lambda i: (0, 0)),
            pl.BlockSpec((_F, n_out), lambda i: (0, 0)),
            pl.BlockSpec((_NAGG, n_out), lambda i: (0, 0)),
            pl.BlockSpec((1, n_out), lambda i: (0, 0)),
        ],
        out_specs=pl.BlockSpec((_BB, _V, n_out), lambda i: (i, 0, 0)),
        out_shape=jax.ShapeDtypeStruct((_B, _V, n_out), jnp.float32),
    )(x, agg, wa, wb, wc, b)


@jax.jit
def kernel(x, W_slr, b_slr, W_out, b_out):
    f = _F
    s, lr = _phase_a(x, W_slr[:f], W_slr[f:], b_slr.reshape(1, -1))
    agg = _phase_b(s, lr)
    wc = jnp.concatenate(
        [W_out[2 * f:2 * f + _NLR] / float(_K), W_out[2 * f + _NLR:]], axis=0)
    out = _phase_c(x, agg, W_out[:f], W_out[f:2 * f], wc,
                   b_out.reshape(1, -1))
    return out


# speedup vs baseline: 1.1755x; 1.0078x over previous
import functools

import jax
import jax.numpy as jnp
from jax import lax
from jax.experimental import pallas as pl
from jax.experimental.pallas import tpu as pltpu
from jax.experimental.pallas import tpu_sc as plsc

_B, _V, _F = 64, 512, 128
_NS, _NLR, _K = 4, 32, 16
_NSLR = _NS + _NLR
_NAGG = 2 * _NLR


_BB = 16


def _phase_a_body(x_ref, wx_ref, wm_ref, b_ref, s_ref, lr_ref):
    for q in range(_BB):
        xb = x_ref[q]
        xm = jnp.mean(xb, axis=0, keepdims=True)
        y = jnp.dot(xb, wx_ref[...], preferred_element_type=jnp.float32)
        y = y + jnp.dot(xm, wm_ref[...],
                        preferred_element_type=jnp.float32)
        y = jnp.maximum(y + b_ref[...], 0.0)
        s_ref[q] = y[:, :_NS]
        lr_ref[q] = y[:, _NS:]


def _phase_a(x, wx, wm, b):
    return pl.pallas_call(
        _phase_a_body,
        grid=(_B // _BB,),
        in_specs=[
            pl.BlockSpec((_BB, _V, _F), lambda i: (i, 0, 0)),
            pl.BlockSpec((_F, _NSLR), lambda i: (0, 0)),
            pl.BlockSpec((_F, _NSLR), lambda i: (0, 0)),
            pl.BlockSpec((1, _NSLR), lambda i: (0, 0)),
        ],
        out_specs=[
            pl.BlockSpec((_BB, _V, _NS), lambda i: (i, 0, 0)),
            pl.BlockSpec((_BB, _V, _NLR), lambda i: (i, 0, 0)),
        ],
        out_shape=[
            jax.ShapeDtypeStruct((_B, _V, _NS), jnp.float32),
            jax.ShapeDtypeStruct((_B, _V, _NLR), jnp.float32),
        ],
    )(x, wx, wm, b)


def _phase_b_body(s_hbm, lr_hbm, agg_hbm,
                  s_v, lr_v, out_v, dbuf_v,
                  bltj_v, bltd_v, beqj_v):
    info = plsc.get_sparse_core_info()
    nc = info.num_cores
    wid = lax.axis_index("s") * nc + lax.axis_index("c")
    iota = jnp.arange(16, dtype=jnp.int32)
    inf16 = jnp.full((16,), jnp.inf, jnp.float32)
    zero16i = jnp.zeros((16,), jnp.int32)

    def one_batch(b):
        pltpu.sync_copy(s_hbm.at[b], s_v.at[pl.ds(0, _V * _NS)])
        pltpu.sync_copy(lr_hbm.at[b], lr_v.at[pl.ds(0, _V * _NLR)])

        def finish_row(i, best_d, best_j, d17v, dbase):
            t = jnp.max(best_d)
            t17 = jnp.min(d17v)

            def fixup(_):
                c_lt = jnp.sum((best_d < t).astype(jnp.int32))

                def fchunk(c, offs):
                    d = dbuf_v[pl.ds(dbase + c * 16, 16)]
                    jv = iota + c * 16

                    def compact(offs):
                        off_lt, off_eq = offs
                        mlt = d < t
                        meq = d == t
                        plsc.store_compressed(bltj_v.at[pl.ds(off_lt, 16)],
                                              jv, mask=mlt)
                        plsc.store_compressed(bltd_v.at[pl.ds(off_lt, 16)],
                                              d, mask=mlt)
                        plsc.store_compressed(beqj_v.at[pl.ds(off_eq, 16)],
                                              jv, mask=meq)
                        off_lt = off_lt + jnp.sum(mlt.astype(jnp.int32))
                        off_eq = off_eq + jnp.sum(meq.astype(jnp.int32))
                        return off_lt, off_eq

                    cnt = plsc.all_reduce_population_count(d <= t)
                    return lax.cond(cnt[0] > 0, compact, lambda oo: oo, offs)

                lax.fori_loop(0, 32, fchunk,
                              (jnp.int32(0), jnp.int32(0)))
                mk = iota < c_lt
                ja = plsc.load_gather(bltj_v, [iota])
                da = plsc.load_gather(bltd_v, [iota])
                jb = plsc.load_gather(
                    beqj_v, [jnp.maximum(iota - c_lt, 0)])
                return (jnp.where(mk, ja, jb),
                        jnp.where(mk, da, t))

            j_sel, d_sel = lax.cond(
                t17 == t, fixup, lambda _: (best_j, best_d), None)
            w = jnp.exp(-10.0 * d_sel)

            z = jnp.zeros((16,), jnp.float32)
            m0, m1, x0, x1 = z, z, z, z
            for k in range(16):
                jk = j_sel[k]
                wk = w[k]
                r0 = wk * lr_v[pl.ds(jk * _NLR, 16)]
                r1 = wk * lr_v[pl.ds(jk * _NLR + 16, 16)]
                m0 = m0 + r0
                m1 = m1 + r1
                x0 = jnp.maximum(x0, r0)
                x1 = jnp.maximum(x1, r1)
            out_v[pl.ds(i * _NAGG, 16)] = m0
            out_v[pl.ds(i * _NAGG + 16, 16)] = m1
            out_v[pl.ds(i * _NAGG + 32, 16)] = x0
            out_v[pl.ds(i * _NAGG + 48, 16)] = x1

        n_rows = 4

        def row_group(p, _):
            i0 = p * n_rows
            srows = [s_v[pl.ds((i0 + 4 * q) * _NS, 16)]
                     for q in range(n_rows // 4)]
            si = [[srows[r // 4][(r % 4) * _NS + c] for c in range(_NS)]
                  for r in range(n_rows)]

            def chunk(c, carry):
                jv = iota + c * 16
                jv4 = jv * _NS
                g = [plsc.load_gather(s_v, [jv4 + cc]) for cc in range(_NS)]
                nxt = []
                for r in range(n_rows):
                    bd, bj, d17 = carry[3 * r:3 * r + 3]
                    e = [g[cc] - si[r][cc] for cc in range(_NS)]
                    d = e[0] * e[0] + e[1] * e[1] + e[2] * e[2] + e[3] * e[3]
                    dbuf_v[pl.ds(r * _V + c * 16, 16)] = d
                    ds, js = plsc.sort_key_val(d, jv)
                    rb = lax.rev(bd, (0,))
                    rbj = lax.rev(bj, (0,))
                    take = ds < rb
                    nd = jnp.where(take, ds, rb)
                    nj = jnp.where(take, js, rbj)
                    d17 = jnp.minimum(d17, jnp.where(take, rb, ds))
                    bd, bj = plsc.sort_key_val(nd, nj)
                    nxt += [bd, bj, d17]
                return tuple(nxt)

            res = lax.fori_loop(
                0, 32, chunk, (inf16, zero16i, inf16) * n_rows)
            for r in range(n_rows):
                finish_row(i0 + r, res[3 * r], res[3 * r + 1],
                           res[3 * r + 2], r * _V)
            return 0

        lax.fori_loop(0, _V // n_rows, row_group, 0)
        pltpu.sync_copy(out_v, agg_hbm.at[b])

    for bi in range(_B // 32):
        one_batch(wid * (_B // 32) + bi)


def _phase_b(s, lr):
    mesh = plsc.VectorSubcoreMesh(core_axis_name="c", subcore_axis_name="s")
    f = pl.kernel(
        _phase_b_body,
        out_type=jax.ShapeDtypeStruct((_B, _V * _NAGG), jnp.float32),
        mesh=mesh,
        compiler_params=pltpu.CompilerParams(needs_layout_passes=False),
        scratch_types=[
            pltpu.VMEM((_V * _NS + 16,), jnp.float32),
            pltpu.VMEM((_V * _NLR + 16,), jnp.float32),
            pltpu.VMEM((_V * _NAGG,), jnp.float32),
            pltpu.VMEM((8 * _V,), jnp.float32),
            pltpu.VMEM((32,), jnp.int32),
            pltpu.VMEM((32,), jnp.float32),
            pltpu.VMEM((_V + 16,), jnp.int32),
        ],
    )
    agg = f(s.reshape(_B, _V * _NS), lr.reshape(_B, _V * _NLR))
    return agg.reshape(_B, _V, _NAGG)


def _phase_c_body(x_ref, agg_ref, wa_ref, wb_ref, wc_ref, b_ref, o_ref):
    for q in range(_BB):
        xb = x_ref[q]
        xm = jnp.mean(xb, axis=0, keepdims=True)
        acc = jnp.dot(xb, wa_ref[...], preferred_element_type=jnp.float32)
        acc = acc + jnp.dot(xm, wb_ref[...],
                            preferred_element_type=jnp.float32)
        acc = acc + jnp.dot(agg_ref[q], wc_ref[...],
                            preferred_element_type=jnp.float32)
        o_ref[q] = jnp.maximum(acc + b_ref[...], 0.0)


def _phase_c(x, agg, wa, wb, wc, b):
    n_out = wa.shape[1]
    return pl.pallas_call(
        _phase_c_body,
        grid=(_B // _BB,),
        in_specs=[
            pl.BlockSpec((_BB, _V, _F), lambda i: (i, 0, 0)),
            pl.BlockSpec((_BB, _V, _NAGG), lambda i: (i, 0, 0)),
            pl.BlockSpec((_F, n_out), lambda i: (0, 0)),
            pl.BlockSpec((_F, n_out), lambda i: (0, 0)),
            pl.BlockSpec((_NAGG, n_out), lambda i: (0, 0)),
            pl.BlockSpec((1, n_out), lambda i: (0, 0)),
        ],
        out_specs=pl.BlockSpec((_BB, _V, n_out), lambda i: (i, 0, 0)),
        out_shape=jax.ShapeDtypeStruct((_B, _V, n_out), jnp.float32),
    )(x, agg, wa, wb, wc, b)


@jax.jit
def kernel(x, W_slr, b_slr, W_out, b_out):
    f = _F
    s, lr = _phase_a(x, W_slr[:f], W_slr[f:], b_slr.reshape(1, -1))
    agg = _phase_b(s, lr)
    wc = jnp.concatenate(
        [W_out[2 * f:2 * f + _NLR] / float(_K), W_out[2 * f + _NLR:]], axis=0)
    out = _phase_c(x, agg, W_out[:f], W_out[f:2 * f], wc,
                   b_out.reshape(1, -1))
    return out


# speedup vs baseline: 1.1761x; 1.0005x over previous
import functools

import jax
import jax.numpy as jnp
from jax import lax
from jax.experimental import pallas as pl
from jax.experimental.pallas import tpu as pltpu
from jax.experimental.pallas import tpu_sc as plsc

_B, _V, _F = 64, 512, 128
_NS, _NLR, _K = 4, 32, 16
_NSLR = _NS + _NLR
_NAGG = 2 * _NLR


_BB = 32


def _phase_a_body(x_ref, wx_ref, wm_ref, b_ref, s_ref, lr_ref):
    for q in range(_BB):
        xb = x_ref[q]
        xm = jnp.mean(xb, axis=0, keepdims=True)
        y = jnp.dot(xb, wx_ref[...], preferred_element_type=jnp.float32)
        y = y + jnp.dot(xm, wm_ref[...],
                        preferred_element_type=jnp.float32)
        y = jnp.maximum(y + b_ref[...], 0.0)
        s_ref[q] = y[:, :_NS]
        lr_ref[q] = y[:, _NS:]


def _phase_a(x, wx, wm, b):
    return pl.pallas_call(
        _phase_a_body,
        grid=(_B // _BB,),
        in_specs=[
            pl.BlockSpec((_BB, _V, _F), lambda i: (i, 0, 0)),
            pl.BlockSpec((_F, _NSLR), lambda i: (0, 0)),
            pl.BlockSpec((_F, _NSLR), lambda i: (0, 0)),
            pl.BlockSpec((1, _NSLR), lambda i: (0, 0)),
        ],
        out_specs=[
            pl.BlockSpec((_BB, _V, _NS), lambda i: (i, 0, 0)),
            pl.BlockSpec((_BB, _V, _NLR), lambda i: (i, 0, 0)),
        ],
        out_shape=[
            jax.ShapeDtypeStruct((_B, _V, _NS), jnp.float32),
            jax.ShapeDtypeStruct((_B, _V, _NLR), jnp.float32),
        ],
    )(x, wx, wm, b)


def _phase_b_body(s_hbm, lr_hbm, agg_hbm,
                  s_v, lr_v, out_v, dbuf_v,
                  bltj_v, bltd_v, beqj_v):
    info = plsc.get_sparse_core_info()
    nc = info.num_cores
    wid = lax.axis_index("s") * nc + lax.axis_index("c")
    iota = jnp.arange(16, dtype=jnp.int32)
    inf16 = jnp.full((16,), jnp.inf, jnp.float32)
    zero16i = jnp.zeros((16,), jnp.int32)

    def one_batch(b):
        pltpu.sync_copy(s_hbm.at[b], s_v.at[pl.ds(0, _V * _NS)])
        pltpu.sync_copy(lr_hbm.at[b], lr_v.at[pl.ds(0, _V * _NLR)])

        def finish_row(i, best_d, best_j, d17v, dbase):
            t = jnp.max(best_d)
            t17 = jnp.min(d17v)

            def fixup(_):
                c_lt = jnp.sum((best_d < t).astype(jnp.int32))

                def fchunk(c, offs):
                    d = dbuf_v[pl.ds(dbase + c * 16, 16)]
                    jv = iota + c * 16

                    def compact(offs):
                        off_lt, off_eq = offs
                        mlt = d < t
                        meq = d == t
                        plsc.store_compressed(bltj_v.at[pl.ds(off_lt, 16)],
                                              jv, mask=mlt)
                        plsc.store_compressed(bltd_v.at[pl.ds(off_lt, 16)],
                                              d, mask=mlt)
                        plsc.store_compressed(beqj_v.at[pl.ds(off_eq, 16)],
                                              jv, mask=meq)
                        off_lt = off_lt + jnp.sum(mlt.astype(jnp.int32))
                        off_eq = off_eq + jnp.sum(meq.astype(jnp.int32))
                        return off_lt, off_eq

                    cnt = plsc.all_reduce_population_count(d <= t)
                    return lax.cond(cnt[0] > 0, compact, lambda oo: oo, offs)

                lax.fori_loop(0, 32, fchunk,
                              (jnp.int32(0), jnp.int32(0)))
                mk = iota < c_lt
                ja = plsc.load_gather(bltj_v, [iota])
                da = plsc.load_gather(bltd_v, [iota])
                jb = plsc.load_gather(
                    beqj_v, [jnp.maximum(iota - c_lt, 0)])
                return (jnp.where(mk, ja, jb),
                        jnp.where(mk, da, t))

            j_sel, d_sel = lax.cond(
                t17 == t, fixup, lambda _: (best_j, best_d), None)
            w = jnp.exp(-10.0 * d_sel)

            z = jnp.zeros((16,), jnp.float32)
            m0, m1, x0, x1 = z, z, z, z
            for k in range(16):
                jk = j_sel[k]
                wk = w[k]
                r0 = wk * lr_v[pl.ds(jk * _NLR, 16)]
                r1 = wk * lr_v[pl.ds(jk * _NLR + 16, 16)]
                m0 = m0 + r0
                m1 = m1 + r1
                x0 = jnp.maximum(x0, r0)
                x1 = jnp.maximum(x1, r1)
            out_v[pl.ds(i * _NAGG, 16)] = m0
            out_v[pl.ds(i * _NAGG + 16, 16)] = m1
            out_v[pl.ds(i * _NAGG + 32, 16)] = x0
            out_v[pl.ds(i * _NAGG + 48, 16)] = x1

        n_rows = 4

        def row_group(p, _):
            i0 = p * n_rows
            srows = [s_v[pl.ds((i0 + 4 * q) * _NS, 16)]
                     for q in range(n_rows // 4)]
            si = [[srows[r // 4][(r % 4) * _NS + c] for c in range(_NS)]
                  for r in range(n_rows)]

            def chunk(c, carry):
                jv = iota + c * 16
                jv4 = jv * _NS
                g = [plsc.load_gather(s_v, [jv4 + cc]) for cc in range(_NS)]
                nxt = []
                for r in range(n_rows):
                    bd, bj, d17 = carry[3 * r:3 * r + 3]
                    e = [g[cc] - si[r][cc] for cc in range(_NS)]
                    d = e[0] * e[0] + e[1] * e[1] + e[2] * e[2] + e[3] * e[3]
                    dbuf_v[pl.ds(r * _V + c * 16, 16)] = d
                    ds, js = plsc.sort_key_val(d, jv)
                    rb = lax.rev(bd, (0,))
                    rbj = lax.rev(bj, (0,))
                    take = ds < rb
                    nd = jnp.where(take, ds, rb)
                    nj = jnp.where(take, js, rbj)
                    d17 = jnp.minimum(d17, jnp.where(take, rb, ds))
                    bd, bj = plsc.sort_key_val(nd, nj)
                    nxt += [bd, bj, d17]
                return tuple(nxt)

            res = lax.fori_loop(
                0, 32, chunk, (inf16, zero16i, inf16) * n_rows)
            for r in range(n_rows):
                finish_row(i0 + r, res[3 * r], res[3 * r + 1],
                           res[3 * r + 2], r * _V)
            return 0

        lax.fori_loop(0, _V // n_rows, row_group, 0)
        pltpu.sync_copy(out_v, agg_hbm.at[b])

    for bi in range(_B // 32):
        one_batch(wid * (_B // 32) + bi)


def _phase_b(s, lr):
    mesh = plsc.VectorSubcoreMesh(core_axis_name="c", subcore_axis_name="s")
    f = pl.kernel(
        _phase_b_body,
        out_type=jax.ShapeDtypeStruct((_B, _V * _NAGG), jnp.float32),
        mesh=mesh,
        compiler_params=pltpu.CompilerParams(needs_layout_passes=False),
        scratch_types=[
            pltpu.VMEM((_V * _NS + 16,), jnp.float32),
            pltpu.VMEM((_V * _NLR + 16,), jnp.float32),
            pltpu.VMEM((_V * _NAGG,), jnp.float32),
            pltpu.VMEM((8 * _V,), jnp.float32),
            pltpu.VMEM((32,), jnp.int32),
            pltpu.VMEM((32,), jnp.float32),
            pltpu.VMEM((_V + 16,), jnp.int32),
        ],
    )
    agg = f(s.reshape(_B, _V * _NS), lr.reshape(_B, _V * _NLR))
    return agg.reshape(_B, _V, _NAGG)


def _phase_c_body(x_ref, agg_ref, wa_ref, wb_ref, wc_ref, b_ref, o_ref):
    for q in range(_BB):
        xb = x_ref[q]
        xm = jnp.mean(xb, axis=0, keepdims=True)
        acc = jnp.dot(xb, wa_ref[...], preferred_element_type=jnp.float32)
        acc = acc + jnp.dot(xm, wb_ref[...],
                            preferred_element_type=jnp.float32)
        acc = acc + jnp.dot(agg_ref[q], wc_ref[...],
                            preferred_element_type=jnp.float32)
        o_ref[q] = jnp.maximum(acc + b_ref[...], 0.0)


def _phase_c(x, agg, wa, wb, wc, b):
    n_out = wa.shape[1]
    return pl.pallas_call(
        _phase_c_body,
        grid=(_B // _BB,),
        in_specs=[
            pl.BlockSpec((_BB, _V, _F), lambda i: (i, 0, 0)),
            pl.BlockSpec((_BB, _V, _NAGG), lambda i: (i, 0, 0)),
            pl.BlockSpec((_F, n_out), lambda i: (0, 0)),
            pl.BlockSpec((_F, n_out), lambda i: (0, 0)),
            pl.BlockSpec((_NAGG, n_out), lambda i: (0, 0)),
            pl.BlockSpec((1, n_out), lambda i: (0, 0)),
        ],
        out_specs=pl.BlockSpec((_BB, _V, n_out), lambda i: (i, 0, 0)),
        out_shape=jax.ShapeDtypeStruct((_B, _V, n_out), jnp.float32),
    )(x, agg, wa, wb, wc, b)


@jax.jit
def kernel(x, W_slr, b_slr, W_out, b_out):
    f = _F
    s, lr = _phase_a(x, W_slr[:f], W_slr[f:], b_slr.reshape(1, -1))
    agg = _phase_b(s, lr)
    wc = jnp.concatenate(
        [W_out[2 * f:2 * f + _NLR] / float(_K), W_out[2 * f + _NLR:]], axis=0)
    out = _phase_c(x, agg, W_out[:f], W_out[f:2 * f], wc,
                   b_out.reshape(1, -1))
    return out
